# Initial kernel scaffold; baseline (speedup 1.0000x reference)
import jax
import jax.numpy as jnp
from jax.experimental import pallas as pl


def kernel(input_ids, table, norm_weight):
    raise NotImplementedError("write your pallas kernel here")


# speedup vs baseline: 1.2253x; 1.2253x over previous
import functools

import jax
import jax.numpy as jnp
from jax import lax
from jax.experimental import pallas as pl
from jax.experimental.pallas import tpu as pltpu
from jax.experimental.pallas import tpu_sc as plsc

VOCAB = 100000
HIDDEN = 128
EPS = 1e-6
LANES = 16
NBLK = HIDDEN // LANES

NC = 2
NS = 16
NW = NC * NS

CHUNK = 128


def _rsqrt(m):
    i = lax.bitcast_convert_type(m, jnp.int32)
    i = 0x5F3759DF - lax.shift_right_logical(i, 1)
    y = lax.bitcast_convert_type(i, jnp.float32)
    h = m * 0.5
    for _ in range(3):
        y = y * (1.5 - h * y * y)
    return y


def _lane_sum_splat(x, scr):
    lanes = lax.iota(jnp.int32, LANES)
    for k in (8, 4, 2, 1):
        scr[...] = x
        x = x + plsc.load_gather(scr, [jnp.bitwise_xor(lanes, k)])
    return x


def _make_kernel(n_rows):
    assert n_rows % (NW * CHUNK) == 0
    rows_per_w = n_rows // NW
    nchunk = rows_per_w // CHUNK
    mesh = plsc.VectorSubcoreMesh(core_axis_name="c", subcore_axis_name="s")

    @functools.partial(
        pl.kernel,
        out_type=jax.ShapeDtypeStruct((n_rows, HIDDEN), jnp.float32),
        mesh=mesh,
        compiler_params=pltpu.CompilerParams(needs_layout_passes=False),
        scratch_types=[
            pltpu.VMEM((nchunk, CHUNK), jnp.int32),
            pltpu.VMEM((HIDDEN,), jnp.float32),
            pltpu.VMEM((CHUNK, HIDDEN), jnp.float32),
            pltpu.VMEM((CHUNK, LANES), jnp.float32),
            pltpu.SemaphoreType.DMA,
        ],
    )
    def k(table_hbm, idx_hbm, w_hbm, out_hbm, idx_v, w_v, buf, scr, sem):
        wid = lax.axis_index("s") * NC + lax.axis_index("c")
        base = wid * rows_per_w
        pltpu.sync_copy(idx_hbm.at[wid], idx_v)
        pltpu.sync_copy(w_hbm, w_v)

        def chunk_body(c, carry):
            pltpu.async_copy(table_hbm.at[idx_v.at[c]], buf, sem).wait()

            def row_body(r, carry2):
                xs = []
                acc = None
                for j in range(NBLK):
                    x = buf[r, pl.ds(j * LANES, LANES)]
                    xs.append(x)
                    acc = x * x if acc is None else acc + x * x
                s = _lane_sum_splat(acc, scr.at[r])
                y = _rsqrt(s * (1.0 / HIDDEN) + EPS)
                for j in range(NBLK):
                    w = w_v[pl.ds(j * LANES, LANES)]
                    buf[r, pl.ds(j * LANES, LANES)] = xs[j] * (y * w)
                return carry2

            lax.fori_loop(0, CHUNK, row_body, 0)
            pltpu.sync_copy(buf, out_hbm.at[pl.ds(base + c * CHUNK, CHUNK)])
            return carry

        lax.fori_loop(0, nchunk, chunk_body, 0)

    return k


def kernel(input_ids, table, norm_weight):
    b, s = input_ids.shape
    n_rows = b * s
    idx3 = input_ids.astype(jnp.int32).reshape(NW, n_rows // (NW * CHUNK), CHUNK)
    out = _make_kernel(n_rows)(table, idx3, norm_weight)
    return out.reshape(b, s, HIDDEN)


# speedup vs baseline: 4.3977x; 3.5891x over previous
import functools

import jax
import jax.numpy as jnp
from jax import lax
from jax.experimental import pallas as pl
from jax.experimental.pallas import tpu as pltpu
from jax.experimental.pallas import tpu_sc as plsc

VOCAB = 100000
HIDDEN = 128
EPS = 1e-6
LANES = 16
NBLK = HIDDEN // LANES

NC = 2
NS = 16
NW = NC * NS

CHUNK = 128
NBUF = 4


def _rsqrt(m):
    i = lax.bitcast_convert_type(m, jnp.int32)
    i = 0x5F3759DF - lax.shift_right_logical(i, 1)
    y = lax.bitcast_convert_type(i, jnp.float32)
    h = m * 0.5
    for _ in range(3):
        y = y * (1.5 - h * y * y)
    return y


def _lane_sum_splat(x, scr):
    lanes = lax.iota(jnp.int32, LANES)
    for k in (8, 4, 2, 1):
        scr[...] = x
        x = x + plsc.load_gather(scr, [jnp.bitwise_xor(lanes, k)])
    return x


def _make_kernel(n_rows):
    assert n_rows % (NW * CHUNK) == 0
    rows_per_w = n_rows // NW
    nchunk = rows_per_w // CHUNK
    mesh = plsc.VectorSubcoreMesh(core_axis_name="c", subcore_axis_name="s")

    @functools.partial(
        pl.kernel,
        out_type=jax.ShapeDtypeStruct((n_rows, HIDDEN), jnp.float32),
        mesh=mesh,
        compiler_params=pltpu.CompilerParams(needs_layout_passes=False),
        scratch_types=[
            pltpu.VMEM((nchunk, CHUNK), jnp.int32),
            pltpu.VMEM((HIDDEN,), jnp.float32),
            pltpu.VMEM((NBUF, CHUNK, HIDDEN), jnp.float32),
            pltpu.VMEM((CHUNK, LANES), jnp.float32),
            pltpu.SemaphoreType.DMA((NBUF,)),
            pltpu.SemaphoreType.DMA((NBUF,)),
        ],
    )
    def k(table_hbm, idx_hbm, w_hbm, out_hbm, idx_v, w_v, bufs, scr, gsem, wsem):
        wid = lax.axis_index("s") * NC + lax.axis_index("c")
        base = wid * rows_per_w
        pltpu.sync_copy(idx_hbm.at[wid], idx_v)
        pltpu.sync_copy(w_hbm, w_v)

        def fire_gather(c):
            b = lax.rem(c, NBUF)
            pltpu.async_copy(table_hbm.at[idx_v.at[c]], bufs.at[b], gsem.at[b])

        fire_gather(0)
        fire_gather(1)

        def step(c, carry):
            b = lax.rem(c, NBUF)
            buf = bufs.at[b]
            pltpu.make_async_copy(table_hbm.at[idx_v.at[c]], buf,
                                  gsem.at[b]).wait()

            @plsc.parallel_loop(0, CHUNK, unroll=4)
            def row_body(r):
                xs = []
                acc = None
                for j in range(NBLK):
                    x = buf[r, pl.ds(j * LANES, LANES)]
                    xs.append(x)
                    acc = x * x if acc is None else acc + x * x
                s = _lane_sum_splat(acc, scr.at[r])
                y = _rsqrt(s * (1.0 / HIDDEN) + EPS)
                for j in range(NBLK):
                    w = w_v[pl.ds(j * LANES, LANES)]
                    buf[r, pl.ds(j * LANES, LANES)] = xs[j] * (y * w)

            out_slice = out_hbm.at[pl.ds(base + c * CHUNK, CHUNK)]
            pltpu.async_copy(buf, out_slice, wsem.at[b])

            @pl.when(c >= NBUF - 2)
            def _():
                bn = lax.rem(c + 2, NBUF)
                pltpu.make_async_copy(
                    bufs.at[bn], out_hbm.at[pl.ds(0, CHUNK)], wsem.at[bn]
                ).wait()

            @pl.when(c + 2 < nchunk)
            def _():
                fire_gather(c + 2)

            return carry

        lax.fori_loop(0, nchunk, step, 0)

        for c in (nchunk - 2, nchunk - 1):
            b = c % NBUF
            pltpu.make_async_copy(
                bufs.at[b], out_hbm.at[pl.ds(0, CHUNK)], wsem.at[b]
            ).wait()

    return k


def kernel(input_ids, table, norm_weight):
    b, s = input_ids.shape
    n_rows = b * s
    idx3 = input_ids.astype(jnp.int32).reshape(NW, n_rows // (NW * CHUNK), CHUNK)
    out = _make_kernel(n_rows)(table, idx3, norm_weight)
    return out.reshape(b, s, HIDDEN)
